# baseline (speedup 1.0000x reference)
import functools
import math

import jax
import jax.numpy as jnp
from jax import lax
from jax.experimental import pallas as pl
from jax.experimental.pallas import tpu as pltpu
from jax.experimental.pallas import tpu_sc as plsc

SMOOTH = 0.1
PAD = 0

BR = 128
BC = 32000
NT = 1408
NW = 32
LANES = 16


def _tc_loss_kernel(tgt_ref, x_ref, o_ref, *, eps, const):
    i = pl.program_id(0)
    j = pl.program_id(1)

    @pl.when(jnp.logical_and(i == 0, j == 0))
    def _():
        o_ref[...] = jnp.zeros((1, 1), jnp.float32)

    x = x_ref[...]
    t = tgt_ref[...].astype(jnp.int32)
    cols = lax.broadcasted_iota(jnp.int32, (BR, BC), 1) + j * BC
    pts = jnp.sum(jnp.where(cols == t, x, 0.0), axis=1)
    rs = jnp.sum(x, axis=1)
    mask = t[:, 0] != PAD
    part = jnp.sum(jnp.where(mask, -eps * rs + (eps - 0.9) * pts, 0.0))
    first = (j == 0).astype(jnp.float32)
    part = part + first * jnp.sum(jnp.where(mask, const + eps * x[:, 0], 0.0))
    o_ref[...] += part.reshape(1, 1)


def _sc_row_kernel(pred_hbm, tgt_hbm, out_hbm, tgt_v, buf0, buf1, accbuf,
                   sem_a, sem_b, *, eps, const, base0, rpw, vocab):
    wid = lax.axis_index("s") * 2 + lax.axis_index("c")
    base = base0 + wid * rpw

    abase = (base // 8) * 8
    off = base - abase
    pltpu.sync_copy(tgt_hbm.at[pl.ds(abase, 3 * 8)], tgt_v.at[pl.ds(0, 3 * 8)])

    zeros16 = jnp.zeros((LANES,), jnp.float32)
    zidx = jnp.zeros((LANES,), jnp.int32)
    lane0 = lax.iota(jnp.int32, LANES) == 0
    nchunk = vocab // (4 * LANES)

    def row_term(buf, l, acc):
        sum_acc, pick_acc = acc

        def ib(k, carry):
            a0, a1 = carry
            o = pl.multiple_of(k * (4 * LANES), 4 * LANES)
            a0 = a0 + buf[pl.ds(o, LANES)]
            a1 = a1 + buf[pl.ds(o + LANES, LANES)]
            a0 = a0 + buf[pl.ds(o + 2 * LANES, LANES)]
            a1 = a1 + buf[pl.ds(o + 3 * LANES, LANES)]
            return a0, a1

        a0, a1 = lax.fori_loop(0, nchunk, ib, (zeros16, zeros16))
        tvl = plsc.load_gather(tgt_v, [jnp.full((LANES,), l + off, jnp.int32)])
        rm = tvl != PAD
        sum_acc = sum_acc + jnp.where(rm, a0 + a1, 0.0)
        ptv = plsc.load_gather(buf, [tvl])
        p0v = plsc.load_gather(buf, [zidx])
        pick = jnp.where(jnp.logical_and(lane0, rm),
                         const + eps * p0v + (eps - 0.9) * ptv, 0.0)
        return sum_acc, pick_acc + pick

    pltpu.async_copy(pred_hbm.at[base], buf0, sem_a)

    def body(g, acc):
        r0 = base + 2 * g
        pltpu.async_copy(pred_hbm.at[r0 + 1], buf1, sem_b)
        pltpu.make_async_copy(pred_hbm.at[r0], buf0, sem_a).wait()
        acc = row_term(buf0, 2 * g, acc)
        nxt = jnp.minimum(r0 + 2, base + rpw - 1)
        pltpu.async_copy(pred_hbm.at[nxt], buf0, sem_a)
        pltpu.make_async_copy(pred_hbm.at[r0 + 1], buf1, sem_b).wait()
        acc = row_term(buf1, 2 * g + 1, acc)
        return acc

    sum_acc, pick_acc = lax.fori_loop(0, rpw // 2, body, (zeros16, zeros16))
    pltpu.make_async_copy(pred_hbm.at[base], buf0, sem_a).wait()

    accbuf[...] = pick_acc - eps * sum_acc
    pltpu.sync_copy(accbuf, out_hbm.at[wid])


def _combine_kernel(tc_ref, sc_ref, o_ref):
    total = jnp.sum(tc_ref[...]) + jnp.sum(sc_ref[...])
    o_ref[...] = total.reshape(1, 1)


def kernel(predicted_tensor, target_tensor):
    B, S, V = predicted_tensor.shape
    N = B * S
    pred = predicted_tensor.reshape(N, V)
    tgt_flat = target_tensor.reshape(N).astype(jnp.int32)
    tgt_col = tgt_flat.reshape(N, 1)

    eps = SMOOTH / (V - 2)
    const = SMOOTH * math.log(eps) + (1.0 - SMOOTH) * math.log(1.0 - SMOOTH)
    rpw = (N - NT) // NW

    tc_part = pl.pallas_call(
        functools.partial(_tc_loss_kernel, eps=eps, const=const),
        grid=(NT // BR, V // BC),
        in_specs=[
            pl.BlockSpec((BR, 1), lambda i, j: (i, 0)),
            pl.BlockSpec((BR, BC), lambda i, j: (i, j)),
        ],
        out_specs=pl.BlockSpec((1, 1), lambda i, j: (0, 0)),
        out_shape=jax.ShapeDtypeStruct((1, 1), jnp.float32),
    )(tgt_col, pred)

    sc_kernel = functools.partial(
        pl.kernel,
        mesh=plsc.VectorSubcoreMesh(core_axis_name="c", subcore_axis_name="s"),
        out_type=jax.ShapeDtypeStruct((NW, LANES), jnp.float32),
        scratch_types=[
            pltpu.VMEM((24,), jnp.int32),
            pltpu.VMEM((V,), jnp.float32),
            pltpu.VMEM((V,), jnp.float32),
            pltpu.VMEM((LANES,), jnp.float32),
            pltpu.SemaphoreType.DMA,
            pltpu.SemaphoreType.DMA,
        ],
        compiler_params=pltpu.CompilerParams(needs_layout_passes=False),
    )(functools.partial(_sc_row_kernel, eps=eps, const=const,
                        base0=NT, rpw=rpw, vocab=V))
    sc_part = sc_kernel(pred, tgt_flat)

    out = pl.pallas_call(
        _combine_kernel,
        out_shape=jax.ShapeDtypeStruct((1, 1), jnp.float32),
    )(tc_part, sc_part)
    return out[0, 0]


# speedup vs baseline: 1.2690x; 1.2690x over previous
import functools
import math

import jax
import jax.numpy as jnp
from jax import lax
from jax.experimental import pallas as pl

SMOOTH = 0.1
PAD = 0

BR = 128
BC = 32000


HALF = BC // 2


def _loss_kernel(tgt_ref, xa_ref, xb_ref, o_ref, *, eps, const):
    i = pl.program_id(0)

    @pl.when(i == 0)
    def _():
        o_ref[...] = jnp.zeros((1, 1), jnp.float32)

    xa = xa_ref[...]
    xb = xb_ref[...]
    t = tgt_ref[...].astype(jnp.int32)
    cols = lax.broadcasted_iota(jnp.int32, (BR, HALF), 1)
    pts = jnp.sum(jnp.where(cols == t, xa, 0.0), axis=1)
    pts = pts + jnp.sum(jnp.where(cols + HALF == t, xb, 0.0), axis=1)
    rs = jnp.sum(xa, axis=1) + jnp.sum(xb, axis=1)
    mask = t[:, 0] != PAD
    part = jnp.sum(jnp.where(mask, -eps * rs + (eps - 0.9) * pts, 0.0))
    part = part + jnp.sum(jnp.where(mask, const + eps * xa[:, 0], 0.0))
    o_ref[...] += part.reshape(1, 1)


def kernel(predicted_tensor, target_tensor):
    B, S, V = predicted_tensor.shape
    N = B * S
    pred = predicted_tensor.reshape(N, V)
    tgt = target_tensor.reshape(N, 1).astype(jnp.int32)

    eps = SMOOTH / (V - 2)
    const = SMOOTH * math.log(eps) + (1.0 - SMOOTH) * math.log(1.0 - SMOOTH)

    out = pl.pallas_call(
        functools.partial(_loss_kernel, eps=eps, const=const),
        grid=(N // BR,),
        in_specs=[
            pl.BlockSpec((BR, 1), lambda i: (i, 0)),
            pl.BlockSpec((BR, HALF), lambda i: (i, 0)),
            pl.BlockSpec((BR, HALF), lambda i: (i, 1)),
        ],
        out_specs=pl.BlockSpec((1, 1), lambda i: (0, 0)),
        out_shape=jax.ShapeDtypeStruct((1, 1), jnp.float32),
    )(tgt, pred, pred)
    return out[0, 0]
